# baseline (speedup 1.0000x reference)
import jax
import jax.numpy as jnp
from jax import lax
from jax.experimental import pallas as pl
from jax.experimental.pallas import tpu as pltpu
from jax.experimental.pallas import tpu_sc as plsc

_NC = 2
_NS = 16
_NW = _NC * _NS
_LANES = 16


def _sc_gather_mean(x2, emb):
  B2, L2 = x2.shape
  L = L2 // 2
  B = B2 * 2
  V, E = emb.shape
  pairs_per_worker = B2 // _NW
  rows_per_worker = 2 * pairs_per_worker
  inv_l = 1.0 / L

  mesh = plsc.VectorSubcoreMesh(core_axis_name="c", subcore_axis_name="s")

  def body(x_hbm, emb_hbm, v_hbm, idx_v, rows0, rows1, rows2, rows3, out_v,
           sem0, sem1, sem2, sem3):
    bufs = (rows0, rows1, rows2, rows3)
    sems = (sem0, sem1, sem2, sem3)
    nbuf = 4
    wid = lax.axis_index("s") * _NC + lax.axis_index("c")
    pbase = wid * pairs_per_worker
    pltpu.sync_copy(x_hbm.at[pl.ds(pbase, pairs_per_worker)], idx_v)

    def reduce_pair(buf, orow):
      for half in range(2):
        lo = half * L
        for c in range(0, E, _LANES):
          sl = pl.ds(c, _LANES)
          acc = [buf[pl.ds(lo + j, 1), sl] for j in range(4)]
          for l in range(4, L - (L % 4), 4):
            for j in range(4):
              acc[j] = acc[j] + buf[pl.ds(lo + l + j, 1), sl]
          for l in range(L - (L % 4), L):
            acc[0] = acc[0] + buf[pl.ds(lo + l, 1), sl]
          total = (acc[0] + acc[1]) + (acc[2] + acc[3])
          out_v[pl.ds(orow + half, 1), sl] = total * inv_l

    for k in range(nbuf - 1):
      pltpu.async_copy(emb_hbm.at[idx_v.at[k]], bufs[k], sems[k])

    @pl.loop(0, pairs_per_worker, step=nbuf)
    def _(p):
      for k in range(nbuf):
        pltpu.make_async_copy(
            emb_hbm.at[idx_v.at[p + k]], bufs[k], sems[k]).wait()

        @pl.when(p + k + nbuf - 1 < pairs_per_worker)
        def _():
          nxt = (k + nbuf - 1) % nbuf
          pltpu.async_copy(
              emb_hbm.at[idx_v.at[p + k + nbuf - 1]], bufs[nxt], sems[nxt])

        reduce_pair(bufs[k], 2 * (p + k))

    pltpu.sync_copy(out_v, v_hbm.at[pl.ds(2 * pbase, rows_per_worker)])

  kern = pl.kernel(
      body,
      out_type=jax.ShapeDtypeStruct((B, E), jnp.float32),
      mesh=mesh,
      scratch_types=[
          pltpu.VMEM((pairs_per_worker, L2), jnp.int32),
          pltpu.VMEM((L2, E), jnp.float32),
          pltpu.VMEM((L2, E), jnp.float32),
          pltpu.VMEM((L2, E), jnp.float32),
          pltpu.VMEM((L2, E), jnp.float32),
          pltpu.VMEM((rows_per_worker, E), jnp.float32),
          pltpu.SemaphoreType.DMA,
          pltpu.SemaphoreType.DMA,
          pltpu.SemaphoreType.DMA,
          pltpu.SemaphoreType.DMA,
      ],
  )
  return kern(x2, emb)


def _mm_body(w_ref, v_ref, b_ref, o_ref):
  acc = lax.dot_general(
      w_ref[...], v_ref[...],
      (((1,), (1,)), ((), ())),
      preferred_element_type=jnp.float32,
  )
  o_ref[...] = acc + b_ref[...]


def _mm_body_alias(w_ref, v_ref, b_ref, prev_ref, o_ref):
  del prev_ref
  _mm_body(w_ref, v_ref, b_ref, o_ref)


def _tc_project_chunk(w16, v16, b2d, prev, ci, nchunks, tile_n):
  Bc, E = v16.shape
  V = w16.shape[0]
  grid = (pl.cdiv(V, tile_n),)
  in_specs = [
      pl.BlockSpec((tile_n, E), lambda n: (n, 0)),
      pl.BlockSpec((Bc, E), lambda n: (0, 0)),
      pl.BlockSpec((tile_n, 1), lambda n: (n, 0)),
  ]
  out_spec = pl.BlockSpec((tile_n, Bc), lambda n, ci=ci: (n, ci))
  out_shape = jax.ShapeDtypeStruct((V, Bc * nchunks), jnp.float32)
  if prev is None:
    return pl.pallas_call(
        _mm_body,
        grid=grid,
        in_specs=in_specs,
        out_specs=out_spec,
        out_shape=out_shape,
    )(w16, v16, b2d)
  return pl.pallas_call(
      _mm_body_alias,
      grid=grid,
      in_specs=in_specs + [pl.BlockSpec(memory_space=pl.ANY)],
      out_specs=out_spec,
      out_shape=out_shape,
      input_output_aliases={3: 0},
  )(w16, v16, b2d, prev)


_NCHUNKS = 2


def kernel(x, emb, W, b):
  V, E = emb.shape
  B, L = x.shape
  x2 = x.reshape(B // 2, 2 * L)
  w16 = W.astype(jnp.bfloat16)
  b2d = b.reshape(V, 1)
  pairs_per_chunk = (B // _NCHUNKS) // 2
  out = None
  for ci in range(_NCHUNKS):
    xc = lax.slice_in_dim(x2, ci * pairs_per_chunk, (ci + 1) * pairs_per_chunk)
    vc16 = _sc_gather_mean(xc, emb).astype(jnp.bfloat16)
    out = _tc_project_chunk(w16, vc16, b2d, out, ci, _NCHUNKS, tile_n=1024)
  return out.T


# speedup vs baseline: 1.0324x; 1.0324x over previous
import jax
import jax.numpy as jnp
from jax import lax
from jax.experimental import pallas as pl
from jax.experimental.pallas import tpu as pltpu
from jax.experimental.pallas import tpu_sc as plsc

_NC = 2
_NS = 16
_NW = _NC * _NS
_LANES = 16


def _sc_gather_mean(x2, emb):
  B2, L2 = x2.shape
  L = L2 // 2
  B = B2 * 2
  V, E = emb.shape
  pairs_per_worker = B2 // _NW
  rows_per_worker = 2 * pairs_per_worker
  inv_l = 1.0 / L

  mesh = plsc.VectorSubcoreMesh(core_axis_name="c", subcore_axis_name="s")

  def body(x_hbm, emb_hbm, v_hbm, idx_v, rows0, rows1, rows2, rows3, out_v,
           sem0, sem1, sem2, sem3):
    bufs = (rows0, rows1, rows2, rows3)
    sems = (sem0, sem1, sem2, sem3)
    nbuf = 4
    wid = lax.axis_index("s") * _NC + lax.axis_index("c")
    pbase = wid * pairs_per_worker
    pltpu.sync_copy(x_hbm.at[pl.ds(pbase, pairs_per_worker)], idx_v)

    def reduce_pair(buf, orow):
      for half in range(2):
        lo = half * L
        for c in range(0, E, _LANES):
          sl = pl.ds(c, _LANES)
          acc = [buf[pl.ds(lo + j, 1), sl] for j in range(4)]
          for l in range(4, L - (L % 4), 4):
            for j in range(4):
              acc[j] = acc[j] + buf[pl.ds(lo + l + j, 1), sl]
          for l in range(L - (L % 4), L):
            acc[0] = acc[0] + buf[pl.ds(lo + l, 1), sl]
          total = (acc[0] + acc[1]) + (acc[2] + acc[3])
          out_v[pl.ds(orow + half, 1), sl] = total * inv_l

    for k in range(nbuf - 1):
      pltpu.async_copy(emb_hbm.at[idx_v.at[k]], bufs[k], sems[k])

    @pl.loop(0, pairs_per_worker, step=nbuf)
    def _(p):
      for k in range(nbuf):
        pltpu.make_async_copy(
            emb_hbm.at[idx_v.at[p + k]], bufs[k], sems[k]).wait()

        @pl.when(p + k + nbuf - 1 < pairs_per_worker)
        def _():
          nxt = (k + nbuf - 1) % nbuf
          pltpu.async_copy(
              emb_hbm.at[idx_v.at[p + k + nbuf - 1]], bufs[nxt], sems[nxt])

        reduce_pair(bufs[k], 2 * (p + k))

    pltpu.sync_copy(out_v, v_hbm.at[pl.ds(2 * pbase, rows_per_worker)])

  kern = pl.kernel(
      body,
      out_type=jax.ShapeDtypeStruct((B, E), jnp.float32),
      mesh=mesh,
      scratch_types=[
          pltpu.VMEM((pairs_per_worker, L2), jnp.int32),
          pltpu.VMEM((L2, E), jnp.float32),
          pltpu.VMEM((L2, E), jnp.float32),
          pltpu.VMEM((L2, E), jnp.float32),
          pltpu.VMEM((L2, E), jnp.float32),
          pltpu.VMEM((rows_per_worker, E), jnp.float32),
          pltpu.SemaphoreType.DMA,
          pltpu.SemaphoreType.DMA,
          pltpu.SemaphoreType.DMA,
          pltpu.SemaphoreType.DMA,
      ],
  )
  return kern(x2, emb)


def _mm_body(w_ref, v_ref, b_ref, o_ref):
  acc = lax.dot_general(
      w_ref[...], v_ref[...],
      (((1,), (1,)), ((), ())),
      preferred_element_type=jnp.float32,
  )
  o_ref[...] = acc + b_ref[...]


def _mm_body_alias(w_ref, v_ref, b_ref, prev_ref, o_ref):
  del prev_ref
  _mm_body(w_ref, v_ref, b_ref, o_ref)


def _tc_project_chunk(w16, v16, b2d, prev, ci, nchunks, tile_n):
  Bc, E = v16.shape
  V = w16.shape[0]
  grid = (pl.cdiv(V, tile_n),)
  in_specs = [
      pl.BlockSpec((tile_n, E), lambda n: (n, 0)),
      pl.BlockSpec((Bc, E), lambda n: (0, 0)),
      pl.BlockSpec((tile_n, 1), lambda n: (n, 0)),
  ]
  out_spec = pl.BlockSpec((tile_n, Bc), lambda n, ci=ci: (n, ci))
  out_shape = jax.ShapeDtypeStruct((V, Bc * nchunks), jnp.float32)
  if prev is None:
    return pl.pallas_call(
        _mm_body,
        grid=grid,
        in_specs=in_specs,
        out_specs=out_spec,
        out_shape=out_shape,
    )(w16, v16, b2d)
  return pl.pallas_call(
      _mm_body_alias,
      grid=grid,
      in_specs=in_specs + [pl.BlockSpec(memory_space=pl.ANY)],
      out_specs=out_spec,
      out_shape=out_shape,
      input_output_aliases={3: 0},
  )(w16, v16, b2d, prev)


_NCHUNKS = 1


def kernel(x, emb, W, b):
  V, E = emb.shape
  B, L = x.shape
  x2 = x.reshape(B // 2, 2 * L)
  w16 = W.astype(jnp.bfloat16)
  b2d = b.reshape(V, 1)
  pairs_per_chunk = (B // _NCHUNKS) // 2
  out = None
  for ci in range(_NCHUNKS):
    xc = lax.slice_in_dim(x2, ci * pairs_per_chunk, (ci + 1) * pairs_per_chunk)
    vc16 = _sc_gather_mean(xc, emb).astype(jnp.bfloat16)
    out = _tc_project_chunk(w16, vc16, b2d, out, ci, _NCHUNKS, tile_n=1024)
  return out.T
